# baseline (speedup 1.0000x reference)
import functools

import jax
import jax.numpy as jnp
from jax import lax
from jax.experimental import pallas as pl
from jax.experimental.pallas import tpu as pltpu
from jax.experimental.pallas import tpu_sc as plsc

F32 = jnp.float32
NC = 2
NS = 16
CH = 80
BLK = 1000


def _full(shape):
    return pl.BlockSpec(shape, lambda i: tuple(0 for _ in shape))


def _encoder(x, W1, b1, W2, b2, scale_by_sig):
    n, din = x.shape
    h = W1.shape[1]
    grid = n // BLK

    def body(x_ref, W1_ref, b1_ref, W2_ref, b2_ref, out_ref, sig_ref):
        xb = x_ref[...]
        sig = jax.nn.sigmoid(xb[:, din - 1:din] / 10.0)
        h1 = jnp.maximum(
            jnp.dot(xb, W1_ref[...], preferred_element_type=F32) + b1_ref[...],
            0.0)
        if scale_by_sig:
            h1 = h1 * sig
        h2 = jnp.dot(h1, W2_ref[...], preferred_element_type=F32) + b2_ref[...]
        out_ref[0] = h2[:, :128]
        out_ref[1] = h2[:, 128:]
        sig_ref[...] = jnp.broadcast_to(sig, (BLK, 8))

    out, sig = pl.pallas_call(
        body,
        grid=(grid,),
        in_specs=[
            pl.BlockSpec((BLK, din), lambda i: (i, 0)),
            _full((din, h)), _full((1, h)), _full((h, h)), _full((1, h)),
        ],
        out_specs=[
            pl.BlockSpec((2, BLK, 128), lambda i: (0, i, 0)),
            pl.BlockSpec((BLK, 8), lambda i: (i, 0)),
        ],
        out_shape=[
            jax.ShapeDtypeStruct((2, n, 128), F32),
            jax.ShapeDtypeStruct((n, 8), F32),
        ],
    )(x, W1, b1.reshape(1, h), W2, b2.reshape(1, h))
    return out.reshape(2 * n, 128), sig


def _combine(sum_flat, cnt, xdst_flat, Wl, bl, Wr, sig):
    n = cnt.shape[0]
    h = Wl.shape[1]
    grid = n // BLK
    nb = n // BLK
    with_sig = sig is not None

    def body(*refs):
        if with_sig:
            (s0, s1, cnt_ref, x0, x1, Wl_ref, bl_ref, Wr_ref, sg_ref,
             out_ref) = refs
        else:
            s0, s1, cnt_ref, x0, x1, Wl_ref, bl_ref, Wr_ref, out_ref = refs
        inv = 1.0 / jnp.maximum(cnt_ref[...][:, 0:1], 1.0)
        mean = jnp.concatenate([s0[...], s1[...]], axis=1) * inv
        xd = jnp.concatenate([x0[...], x1[...]], axis=1)
        o = (jnp.dot(mean, Wl_ref[...], preferred_element_type=F32)
             + bl_ref[...]
             + jnp.dot(xd, Wr_ref[...], preferred_element_type=F32))
        a = jnp.maximum(o, 0.0)
        if with_sig:
            a = a * sg_ref[...][:, 0:1]
        out_ref[0] = a[:, :128]
        out_ref[1] = a[:, 128:]

    in_specs = [
        pl.BlockSpec((BLK, 128), lambda i: (i, 0)),
        pl.BlockSpec((BLK, 128), lambda i, nb=nb: (nb + i, 0)),
        pl.BlockSpec((BLK, 16), lambda i: (i, 0)),
        pl.BlockSpec((BLK, 128), lambda i: (i, 0)),
        pl.BlockSpec((BLK, 128), lambda i, nb=nb: (nb + i, 0)),
        _full((h, h)), _full((1, h)), _full((h, h)),
    ]
    args = [sum_flat, sum_flat, cnt, xdst_flat, xdst_flat,
            Wl, bl.reshape(1, h), Wr]
    if with_sig:
        in_specs.append(pl.BlockSpec((BLK, 8), lambda i: (i, 0)))
        args.append(sig)

    out = pl.pallas_call(
        body,
        grid=(grid,),
        in_specs=in_specs,
        out_specs=pl.BlockSpec((2, BLK, 128), lambda i: (0, i, 0)),
        out_shape=jax.ShapeDtypeStruct((2, n, 128), F32),
    )(*args)
    return out.reshape(2 * n, 128)


def _pred_pre(feat_flat, Wp, bp):
    n = feat_flat.shape[0] // 2
    h = Wp.shape[1]
    grid = n // BLK
    nb = n // BLK
    with_b = bp is not None

    def body(*refs):
        if with_b:
            f0, f1, W_ref, b_ref, out_ref = refs
        else:
            f0, f1, W_ref, out_ref = refs
        f = jnp.concatenate([f0[...], f1[...]], axis=1)
        o = jnp.dot(f, W_ref[...], preferred_element_type=F32)
        if with_b:
            o = o + b_ref[...]
        ob = o.astype(jnp.bfloat16)
        lo = lax.convert_element_type(
            lax.bitcast_convert_type(ob[:, :h // 2], jnp.uint16), jnp.int32)
        hi = lax.convert_element_type(
            lax.bitcast_convert_type(ob[:, h // 2:], jnp.uint16), jnp.int32)
        out_ref[...] = lo | (hi << 16)

    in_specs = [
        pl.BlockSpec((BLK, 128), lambda i: (i, 0)),
        pl.BlockSpec((BLK, 128), lambda i, nb=nb: (nb + i, 0)),
        _full((2 * h if Wp.shape[0] == 2 * h else Wp.shape[0], h)),
    ]
    args = [feat_flat, feat_flat, Wp]
    if with_b:
        in_specs.append(_full((1, h)))
        args.append(bp.reshape(1, h))

    return pl.pallas_call(
        body,
        grid=(grid,),
        in_specs=in_specs,
        out_specs=pl.BlockSpec((BLK, h // 2), lambda i: (i, 0)),
        out_shape=jax.ShapeDtypeStruct((n, h // 2), jnp.int32),
    )(*args)


_MESH = plsc.VectorSubcoreMesh(core_axis_name="c", subcore_axis_name="s")
_SC_PARAMS = pltpu.CompilerParams(use_tc_tiling_on_sc=False,
                                  needs_layout_passes=False)


@functools.lru_cache(maxsize=None)
def _make_segsum(n_src, n_dst, n_edges, with_counts):
    epw = n_edges // NS
    iters = epw // CH
    rps = (n_dst // NS) // 8 * 8
    tail = n_dst - NS * rps

    if with_counts:
        out_type = [jax.ShapeDtypeStruct((2 * n_dst, 128), F32),
                    jax.ShapeDtypeStruct((n_dst, 16), F32)]
    else:
        out_type = jax.ShapeDtypeStruct((2 * n_dst, 128), F32)

    bedge = 4000
    bchunks = bedge // CH
    nblocks = epw // bedge
    scratch = [
        pltpu.VMEM((bedge,), jnp.int32),
        pltpu.VMEM((bedge,), jnp.int32),
        pltpu.VMEM((CH, 128), F32),
        pltpu.VMEM((CH, 128), F32),
        pltpu.VMEM((CH, 16), F32),
        pltpu.VMEM_SHARED((n_dst, 128), F32),
        pltpu.VMEM_SHARED((n_dst, 16), F32),
        pltpu.SemaphoreType.DMA,
        pltpu.SemaphoreType.DMA,
    ]

    def body(table, src_all, dst, z128, z16, ones, *rest):
        if with_counts:
            (out, cnt_out, sidx, didx, rows0, rows1, ones_v, acc, cacc,
             sem0, sem1) = rest
        else:
            (out, sidx, didx, rows0, rows1, ones_v, acc, cacc,
             sem0, sem1) = rest
        c = lax.axis_index("c")
        s = lax.axis_index("s")

        row0 = pl.multiple_of(s * rps, 8)
        pltpu.sync_copy(z128.at[pl.ds(row0, rps)], acc.at[pl.ds(row0, rps)])
        if with_counts:
            pltpu.sync_copy(z16.at[pl.ds(row0, rps)],
                            cacc.at[pl.ds(row0, rps)])
            pltpu.sync_copy(ones, ones_v)
        if tail:
            @pl.when(s == NS - 1)
            def _():
                pltpu.sync_copy(z128.at[pl.ds(NS * rps, tail)],
                                acc.at[pl.ds(NS * rps, tail)])
                if with_counts:
                    pltpu.sync_copy(z16.at[pl.ds(NS * rps, tail)],
                                    cacc.at[pl.ds(NS * rps, tail)])

        src_base = pl.multiple_of(c * n_edges + s * epw, 8)
        dst_base = pl.multiple_of(s * epw, 8)
        plsc.subcore_barrier()

        slots = ((rows0, sem0, rows1, sem1), (rows1, sem1, rows0, sem0))

        def block_step(blk, carry):
            boff = blk * bedge
            pltpu.sync_copy(src_all.at[pl.ds(src_base + boff, bedge)], sidx)
            pltpu.sync_copy(dst.at[pl.ds(dst_base + boff, bedge)], didx)
            pltpu.async_copy(table.at[sidx.at[pl.ds(0, CH)]], rows0, sem0)

            def pair(p, carry2):
                for b, (rb, sb, rn, sn) in enumerate(slots):
                    i = p * 2 + b
                    ioff = pl.multiple_of(i * CH, 8)

                    @pl.when(i + 1 < bchunks)
                    def _():
                        noff = pl.multiple_of((i + 1) * CH, 8)
                        pltpu.async_copy(
                            table.at[sidx.at[pl.ds(noff, CH)]], rn, sn)

                    pltpu.make_async_copy(
                        table.at[sidx.at[pl.ds(ioff, CH)]], rb, sb).wait()
                    pltpu.sync_copy(rb, acc.at[didx.at[pl.ds(ioff, CH)]],
                                    add=True)
                    if with_counts:
                        @pl.when(c == 0)
                        def _():
                            pltpu.sync_copy(
                                ones_v, cacc.at[didx.at[pl.ds(ioff, CH)]],
                                add=True)
                return carry2

            lax.fori_loop(0, bchunks // 2, pair, 0)
            return carry

        lax.fori_loop(0, nblocks, block_step, 0)
        plsc.subcore_barrier()

        orow0 = pl.multiple_of(c * n_dst + s * rps, 8)
        pltpu.sync_copy(acc.at[pl.ds(row0, rps)], out.at[pl.ds(orow0, rps)])
        if with_counts:
            @pl.when(c == 0)
            def _():
                pltpu.sync_copy(cacc.at[pl.ds(row0, rps)],
                                cnt_out.at[pl.ds(row0, rps)])
        if tail:
            @pl.when(s == NS - 1)
            def _():
                otail = pl.multiple_of(c * n_dst + NS * rps, 8)
                pltpu.sync_copy(acc.at[pl.ds(NS * rps, tail)],
                                out.at[pl.ds(otail, tail)])
                if with_counts:
                    @pl.when(c == 0)
                    def _():
                        pltpu.sync_copy(cacc.at[pl.ds(NS * rps, tail)],
                                        cnt_out.at[pl.ds(NS * rps, tail)])

    return pl.kernel(body, out_type=out_type, mesh=_MESH,
                     scratch_types=scratch, compiler_params=_SC_PARAMS)


@functools.lru_cache(maxsize=None)
def _make_edgegather(n_edges, h):
    epw = n_edges // (NC * NS)
    iters = epw // CH

    hw = h // 2
    scratch = [
        pltpu.VMEM((epw,), jnp.int32),
        pltpu.VMEM((epw,), jnp.int32),
        pltpu.VMEM((CH, hw), jnp.int32),
        pltpu.VMEM((CH, hw), jnp.int32),
        pltpu.VMEM((CH, hw), jnp.int32),
        pltpu.VMEM((CH, hw), jnp.int32),
        pltpu.SemaphoreType.DMA,
        pltpu.SemaphoreType.DMA,
        pltpu.SemaphoreType.DMA,
        pltpu.SemaphoreType.DMA,
    ]

    def body(U, B, src, dst, u_out, b_out,
             sidx, didx, u0, u1, b0, b1, su0, su1, sb0, sb1):
        c = lax.axis_index("c")
        s = lax.axis_index("s")
        base = pl.multiple_of((c * NS + s) * epw, 8)

        pltpu.sync_copy(src.at[pl.ds(base, epw)], sidx)
        pltpu.sync_copy(dst.at[pl.ds(base, epw)], didx)

        pltpu.async_copy(U.at[sidx.at[pl.ds(0, CH)]], u0, su0)
        pltpu.async_copy(B.at[didx.at[pl.ds(0, CH)]], b0, sb0)

        slots = ((u0, su0, b0, sb0, u1, su1, b1, sb1),
                 (u1, su1, b1, sb1, u0, su0, b0, sb0))

        def pair(p, carry):
            for b, (ub, us, bb, bs, un, usn, bn, bsn) in enumerate(slots):
                i = p * 2 + b

                @pl.when(i < iters)
                def _():
                    ioff = pl.multiple_of(i * CH, 8)

                    @pl.when(i + 1 < iters)
                    def _():
                        noff = pl.multiple_of((i + 1) * CH, 8)
                        pltpu.async_copy(
                            U.at[sidx.at[pl.ds(noff, CH)]], un, usn)
                        pltpu.async_copy(
                            B.at[didx.at[pl.ds(noff, CH)]], bn, bsn)

                    pltpu.make_async_copy(
                        U.at[sidx.at[pl.ds(ioff, CH)]], ub, us).wait()
                    pltpu.make_async_copy(
                        B.at[didx.at[pl.ds(ioff, CH)]], bb, bs).wait()
                    ooff = pl.multiple_of(base + ioff, 8)
                    pltpu.sync_copy(ub, u_out.at[pl.ds(ooff, CH)])
                    pltpu.sync_copy(bb, b_out.at[pl.ds(ooff, CH)])
            return carry

        lax.fori_loop(0, (iters + 1) // 2, pair, 0)

    ot = jax.ShapeDtypeStruct((n_edges, h // 2), jnp.int32)
    return pl.kernel(body, out_type=[ot, ot],
                     mesh=_MESH, scratch_types=scratch)


def _edge_reduce(ug, bg, wp2, bp2):
    e, hw = ug.shape
    be = 2048
    grid = e // be

    def unpack_lo(x):
        return lax.bitcast_convert_type(x << 16, F32)

    def unpack_hi(x):
        return lax.bitcast_convert_type(x & jnp.int32(-65536), F32)

    def body(u_ref, b_ref, w_ref, b2_ref, out_ref):
        u = u_ref[...]
        b = b_ref[...]
        w = w_ref[...]
        t_lo = jnp.maximum(unpack_lo(u) + unpack_lo(b), 0.0) * unpack_lo(w)
        t_hi = jnp.maximum(unpack_hi(u) + unpack_hi(b), 0.0) * unpack_hi(w)
        srow = jnp.sum(t_lo + t_hi, axis=1) + b2_ref[0, 0]
        out_ref[0, 0] = 1.0 + 4.0 / (1.0 + jnp.exp(-srow))

    out = pl.pallas_call(
        body,
        grid=(grid,),
        in_specs=[
            pl.BlockSpec((be, hw), lambda i: (i, 0)),
            pl.BlockSpec((be, hw), lambda i: (i, 0)),
            _full((1, hw)), _full((1, 1)),
        ],
        out_specs=pl.BlockSpec((1, 1, be), lambda i: (i, 0, 0)),
        out_shape=jax.ShapeDtypeStruct((grid, 1, be), F32),
    )(ug, bg, wp2, bp2.reshape(1, 1))
    return out.reshape(e)


def kernel(x_user, x_book, edge_index,
           Wu1, bu1, Wu2, bu2, Wb1, bb1, Wb2, bb2,
           c1b_Wl, c1b_bl, c1b_Wr, c1u_Wl, c1u_bl, c1u_Wr,
           c2b_Wl, c2b_bl, c2b_Wr, c2u_Wl, c2u_bl, c2u_Wr,
           Wp1, bp1, Wp2, bp2):
    nu = x_user.shape[0]
    nb = x_book.shape[0]
    e = edge_index.shape[1]
    h = Wu1.shape[1]

    src = edge_index[0]
    dst = edge_index[1]
    src_all = jnp.concatenate([src, src + nu])
    dst_all = jnp.concatenate([dst, dst + nb])
    z128 = jnp.zeros((max(nu, nb), 128), F32)
    z16 = jnp.zeros((max(nu, nb), 16), F32)
    ones16 = jnp.ones((CH, 16), F32)

    uh0, sig = _encoder(x_user, Wu1, bu1, Wu2, bu2, True)
    bh0, _ = _encoder(x_book, Wb1, bb1, Wb2, bb2, False)

    seg_c = _make_segsum(nu, nb, e, True)
    sum_b, cnt_b = seg_c(uh0, src_all, dst, z128, z16, ones16)
    seg_u = _make_segsum(nb, nu, e, True)
    sum_u, cnt_u = seg_u(bh0, dst_all, src, z128, z16, ones16)
    bh1 = _combine(sum_b, cnt_b, bh0, c1b_Wl, c1b_bl, c1b_Wr, None)
    uh1 = _combine(sum_u, cnt_u, uh0, c1u_Wl, c1u_bl, c1u_Wr, sig)

    seg2_c = _make_segsum(nu, nb, e, False)
    sum_b2 = seg2_c(uh1, src_all, dst, z128, z16, ones16)
    seg2_u = _make_segsum(nb, nu, e, False)
    sum_u2 = seg2_u(bh1, dst_all, src, z128, z16, ones16)
    bf = _combine(sum_b2, cnt_b, bh1, c2b_Wl, c2b_bl, c2b_Wr, None)
    uf = _combine(sum_u2, cnt_u, uh1, c2u_Wl, c2u_bl, c2u_Wr, sig)

    U = _pred_pre(uf, Wp1[:h], bp1)
    B = _pred_pre(bf, Wp1[h:], None)
    wb = Wp2.reshape(h).astype(jnp.bfloat16)
    wlo = lax.convert_element_type(
        lax.bitcast_convert_type(wb[:h // 2], jnp.uint16), jnp.int32)
    whi = lax.convert_element_type(
        lax.bitcast_convert_type(wb[h // 2:], jnp.uint16), jnp.int32)
    wpk = (wlo | (whi << 16)).reshape(1, h // 2)

    eh = 163840
    pad = 2 * eh - e
    srcp = jnp.concatenate([src, jnp.zeros((pad,), jnp.int32)])
    dstp = jnp.concatenate([dst, jnp.zeros((pad,), jnp.int32)])
    gather = _make_edgegather(eh, h)
    ug0, bg0 = gather(U, B, srcp[:eh], dstp[:eh])
    ug1, bg1 = gather(U, B, srcp[eh:], dstp[eh:])
    r0 = _edge_reduce(ug0, bg0, wpk, bp2)
    r1 = _edge_reduce(ug1, bg1, wpk, bp2)
    return jnp.concatenate([r0, r1])[:e]


# speedup vs baseline: 1.2930x; 1.2930x over previous
import functools

import jax
import jax.numpy as jnp
from jax import lax
from jax.experimental import pallas as pl
from jax.experimental.pallas import tpu as pltpu
from jax.experimental.pallas import tpu_sc as plsc

F32 = jnp.float32
NC = 2
NS = 16
CH = 80
BLK = 1000


def _full(shape):
    return pl.BlockSpec(shape, lambda i: tuple(0 for _ in shape))


def _encoder(x, W1, b1, W2, b2, scale_by_sig):
    n, din = x.shape
    h = W1.shape[1]
    grid = n // BLK

    def body(x_ref, W1_ref, b1_ref, W2_ref, b2_ref, out_ref, sig_ref):
        xb = x_ref[...]
        sig = jax.nn.sigmoid(xb[:, din - 1:din] / 10.0)
        h1 = jnp.maximum(
            jnp.dot(xb, W1_ref[...], preferred_element_type=F32) + b1_ref[...],
            0.0)
        if scale_by_sig:
            h1 = h1 * sig
        h2 = jnp.dot(h1, W2_ref[...], preferred_element_type=F32) + b2_ref[...]
        out_ref[0] = h2[:, :128]
        out_ref[1] = h2[:, 128:]
        sig_ref[...] = jnp.broadcast_to(sig, (BLK, 8))

    out, sig = pl.pallas_call(
        body,
        grid=(grid,),
        in_specs=[
            pl.BlockSpec((BLK, din), lambda i: (i, 0)),
            _full((din, h)), _full((1, h)), _full((h, h)), _full((1, h)),
        ],
        out_specs=[
            pl.BlockSpec((2, BLK, 128), lambda i: (0, i, 0)),
            pl.BlockSpec((BLK, 8), lambda i: (i, 0)),
        ],
        out_shape=[
            jax.ShapeDtypeStruct((2, n, 128), F32),
            jax.ShapeDtypeStruct((n, 8), F32),
        ],
    )(x, W1, b1.reshape(1, h), W2, b2.reshape(1, h))
    return out.reshape(2 * n, 128), sig


def _combine(sum_flat, cnt, xdst_flat, Wl, bl, Wr, sig):
    n = cnt.shape[0]
    h = Wl.shape[1]
    grid = n // BLK
    nb = n // BLK
    with_sig = sig is not None

    def body(*refs):
        if with_sig:
            (s0, s1, cnt_ref, x0, x1, Wl_ref, bl_ref, Wr_ref, sg_ref,
             out_ref) = refs
        else:
            s0, s1, cnt_ref, x0, x1, Wl_ref, bl_ref, Wr_ref, out_ref = refs
        inv = 1.0 / jnp.maximum(cnt_ref[...][:, 0:1], 1.0)
        mean = jnp.concatenate([s0[...], s1[...]], axis=1) * inv
        xd = jnp.concatenate([x0[...], x1[...]], axis=1)
        o = (jnp.dot(mean, Wl_ref[...], preferred_element_type=F32)
             + bl_ref[...]
             + jnp.dot(xd, Wr_ref[...], preferred_element_type=F32))
        a = jnp.maximum(o, 0.0)
        if with_sig:
            a = a * sg_ref[...][:, 0:1]
        out_ref[0] = a[:, :128]
        out_ref[1] = a[:, 128:]

    in_specs = [
        pl.BlockSpec((BLK, 128), lambda i: (i, 0)),
        pl.BlockSpec((BLK, 128), lambda i, nb=nb: (nb + i, 0)),
        pl.BlockSpec((BLK, 16), lambda i: (i, 0)),
        pl.BlockSpec((BLK, 128), lambda i: (i, 0)),
        pl.BlockSpec((BLK, 128), lambda i, nb=nb: (nb + i, 0)),
        _full((h, h)), _full((1, h)), _full((h, h)),
    ]
    args = [sum_flat, sum_flat, cnt, xdst_flat, xdst_flat,
            Wl, bl.reshape(1, h), Wr]
    if with_sig:
        in_specs.append(pl.BlockSpec((BLK, 8), lambda i: (i, 0)))
        args.append(sig)

    out = pl.pallas_call(
        body,
        grid=(grid,),
        in_specs=in_specs,
        out_specs=pl.BlockSpec((2, BLK, 128), lambda i: (0, i, 0)),
        out_shape=jax.ShapeDtypeStruct((2, n, 128), F32),
    )(*args)
    return out.reshape(2 * n, 128)


def _pred_pre(feat_flat, Wp, bp):
    n = feat_flat.shape[0] // 2
    h = Wp.shape[1]
    grid = n // BLK
    nb = n // BLK
    with_b = bp is not None

    def body(*refs):
        if with_b:
            f0, f1, W_ref, b_ref, out_ref = refs
        else:
            f0, f1, W_ref, out_ref = refs
        f = jnp.concatenate([f0[...], f1[...]], axis=1)
        o = jnp.dot(f, W_ref[...], preferred_element_type=F32)
        if with_b:
            o = o + b_ref[...]
        ob = o.astype(jnp.bfloat16)
        lo = lax.convert_element_type(
            lax.bitcast_convert_type(ob[:, :h // 2], jnp.uint16), jnp.int32)
        hi = lax.convert_element_type(
            lax.bitcast_convert_type(ob[:, h // 2:], jnp.uint16), jnp.int32)
        out_ref[...] = lo | (hi << 16)

    in_specs = [
        pl.BlockSpec((BLK, 128), lambda i: (i, 0)),
        pl.BlockSpec((BLK, 128), lambda i, nb=nb: (nb + i, 0)),
        _full((2 * h if Wp.shape[0] == 2 * h else Wp.shape[0], h)),
    ]
    args = [feat_flat, feat_flat, Wp]
    if with_b:
        in_specs.append(_full((1, h)))
        args.append(bp.reshape(1, h))

    return pl.pallas_call(
        body,
        grid=(grid,),
        in_specs=in_specs,
        out_specs=pl.BlockSpec((BLK, h // 2), lambda i: (i, 0)),
        out_shape=jax.ShapeDtypeStruct((n, h // 2), jnp.int32),
    )(*args)


_MESH = plsc.VectorSubcoreMesh(core_axis_name="c", subcore_axis_name="s")
_SC_PARAMS = pltpu.CompilerParams(use_tc_tiling_on_sc=False,
                                  needs_layout_passes=False)


@functools.lru_cache(maxsize=None)
def _make_segsum(n_src, n_dst, n_edges, with_counts):
    epw = n_edges // NS
    iters = epw // CH
    rps = (n_dst // NS) // 8 * 8
    tail = n_dst - NS * rps

    if with_counts:
        out_type = [jax.ShapeDtypeStruct((2 * n_dst, 128), F32),
                    jax.ShapeDtypeStruct((n_dst, 16), F32)]
    else:
        out_type = jax.ShapeDtypeStruct((2 * n_dst, 128), F32)

    bedge = 4000
    bchunks = bedge // CH
    nblocks = epw // bedge
    scratch = [
        pltpu.VMEM((bedge,), jnp.int32),
        pltpu.VMEM((bedge,), jnp.int32),
        pltpu.VMEM((CH, 128), F32),
        pltpu.VMEM((CH, 128), F32),
        pltpu.VMEM((CH, 16), F32),
        pltpu.VMEM_SHARED((n_dst, 128), F32),
        pltpu.VMEM_SHARED((n_dst, 16), F32),
        pltpu.SemaphoreType.DMA,
        pltpu.SemaphoreType.DMA,
    ]

    def body(table, src_all, dst, z128, z16, ones, *rest):
        if with_counts:
            (out, cnt_out, sidx, didx, rows0, rows1, ones_v, acc, cacc,
             sem0, sem1) = rest
        else:
            (out, sidx, didx, rows0, rows1, ones_v, acc, cacc,
             sem0, sem1) = rest
        c = lax.axis_index("c")
        s = lax.axis_index("s")

        row0 = pl.multiple_of(s * rps, 8)
        pltpu.sync_copy(z128.at[pl.ds(row0, rps)], acc.at[pl.ds(row0, rps)])
        if with_counts:
            pltpu.sync_copy(z16.at[pl.ds(row0, rps)],
                            cacc.at[pl.ds(row0, rps)])
            pltpu.sync_copy(ones, ones_v)
        if tail:
            @pl.when(s == NS - 1)
            def _():
                pltpu.sync_copy(z128.at[pl.ds(NS * rps, tail)],
                                acc.at[pl.ds(NS * rps, tail)])
                if with_counts:
                    pltpu.sync_copy(z16.at[pl.ds(NS * rps, tail)],
                                    cacc.at[pl.ds(NS * rps, tail)])

        src_base = pl.multiple_of(c * n_edges + s * epw, 8)
        dst_base = pl.multiple_of(s * epw, 8)
        plsc.subcore_barrier()

        slots = ((rows0, sem0, rows1, sem1), (rows1, sem1, rows0, sem0))

        def block_step(blk, carry):
            boff = blk * bedge
            pltpu.sync_copy(src_all.at[pl.ds(src_base + boff, bedge)], sidx)
            pltpu.sync_copy(dst.at[pl.ds(dst_base + boff, bedge)], didx)
            pltpu.async_copy(table.at[sidx.at[pl.ds(0, CH)]], rows0, sem0)

            def pair(p, carry2):
                for b, (rb, sb, rn, sn) in enumerate(slots):
                    i = p * 2 + b
                    ioff = pl.multiple_of(i * CH, 8)

                    @pl.when(i + 1 < bchunks)
                    def _():
                        noff = pl.multiple_of((i + 1) * CH, 8)
                        pltpu.async_copy(
                            table.at[sidx.at[pl.ds(noff, CH)]], rn, sn)

                    pltpu.make_async_copy(
                        table.at[sidx.at[pl.ds(ioff, CH)]], rb, sb).wait()
                    pltpu.sync_copy(rb, acc.at[didx.at[pl.ds(ioff, CH)]],
                                    add=True)
                    if with_counts:
                        @pl.when(c == 0)
                        def _():
                            pltpu.sync_copy(
                                ones_v, cacc.at[didx.at[pl.ds(ioff, CH)]],
                                add=True)
                return carry2

            lax.fori_loop(0, bchunks // 2, pair, 0)
            return carry

        lax.fori_loop(0, nblocks, block_step, 0)
        plsc.subcore_barrier()

        orow0 = pl.multiple_of(c * n_dst + s * rps, 8)
        pltpu.sync_copy(acc.at[pl.ds(row0, rps)], out.at[pl.ds(orow0, rps)])
        if with_counts:
            @pl.when(c == 0)
            def _():
                pltpu.sync_copy(cacc.at[pl.ds(row0, rps)],
                                cnt_out.at[pl.ds(row0, rps)])
        if tail:
            @pl.when(s == NS - 1)
            def _():
                otail = pl.multiple_of(c * n_dst + NS * rps, 8)
                pltpu.sync_copy(acc.at[pl.ds(NS * rps, tail)],
                                out.at[pl.ds(otail, tail)])
                if with_counts:
                    @pl.when(c == 0)
                    def _():
                        pltpu.sync_copy(cacc.at[pl.ds(NS * rps, tail)],
                                        cnt_out.at[pl.ds(NS * rps, tail)])

    return pl.kernel(body, out_type=out_type, mesh=_MESH,
                     scratch_types=scratch, compiler_params=_SC_PARAMS)


@functools.lru_cache(maxsize=None)
def _make_edgegather(n_edges, h):
    epw = n_edges // (NC * NS)
    iters = epw // CH

    hw = h // 2
    scratch = [
        pltpu.VMEM((epw,), jnp.int32),
        pltpu.VMEM((epw,), jnp.int32),
        pltpu.VMEM((CH, hw), jnp.int32),
        pltpu.VMEM((CH, hw), jnp.int32),
        pltpu.VMEM((CH, hw), jnp.int32),
        pltpu.VMEM((CH, hw), jnp.int32),
        pltpu.SemaphoreType.DMA,
        pltpu.SemaphoreType.DMA,
        pltpu.SemaphoreType.DMA,
        pltpu.SemaphoreType.DMA,
    ]

    def body(U, B, src, dst, u_out, b_out,
             sidx, didx, u0, u1, b0, b1, su0, su1, sb0, sb1):
        c = lax.axis_index("c")
        s = lax.axis_index("s")
        base = pl.multiple_of((c * NS + s) * epw, 8)

        pltpu.sync_copy(src.at[pl.ds(base, epw)], sidx)
        pltpu.sync_copy(dst.at[pl.ds(base, epw)], didx)

        pltpu.async_copy(U.at[sidx.at[pl.ds(0, CH)]], u0, su0)
        pltpu.async_copy(B.at[didx.at[pl.ds(0, CH)]], b0, sb0)

        slots = ((u0, su0, b0, sb0, u1, su1, b1, sb1),
                 (u1, su1, b1, sb1, u0, su0, b0, sb0))

        def pair(p, carry):
            for b, (ub, us, bb, bs, un, usn, bn, bsn) in enumerate(slots):
                i = p * 2 + b

                @pl.when(i < iters)
                def _():
                    ioff = pl.multiple_of(i * CH, 8)

                    @pl.when(i + 1 < iters)
                    def _():
                        noff = pl.multiple_of((i + 1) * CH, 8)
                        pltpu.async_copy(
                            U.at[sidx.at[pl.ds(noff, CH)]], un, usn)
                        pltpu.async_copy(
                            B.at[didx.at[pl.ds(noff, CH)]], bn, bsn)

                    pltpu.make_async_copy(
                        U.at[sidx.at[pl.ds(ioff, CH)]], ub, us).wait()
                    pltpu.make_async_copy(
                        B.at[didx.at[pl.ds(ioff, CH)]], bb, bs).wait()
                    ooff = pl.multiple_of(base + ioff, 8)
                    pltpu.sync_copy(ub, u_out.at[pl.ds(ooff, CH)])
                    pltpu.sync_copy(bb, b_out.at[pl.ds(ooff, CH)])
            return carry

        lax.fori_loop(0, (iters + 1) // 2, pair, 0)

    ot = jax.ShapeDtypeStruct((n_edges, h // 2), jnp.int32)
    return pl.kernel(body, out_type=[ot, ot],
                     mesh=_MESH, scratch_types=scratch)


def _edge_reduce(ug, bg, wp2, bp2):
    e, hw = ug.shape
    be = 3200
    grid = e // be

    def unpack_lo(x):
        return lax.bitcast_convert_type(x << 16, F32)

    def unpack_hi(x):
        return lax.bitcast_convert_type(x & jnp.int32(-65536), F32)

    def body(u_ref, b_ref, w_ref, b2_ref, out_ref):
        u = u_ref[...]
        b = b_ref[...]
        w = w_ref[...]
        t_lo = jnp.maximum(unpack_lo(u) + unpack_lo(b), 0.0) * unpack_lo(w)
        t_hi = jnp.maximum(unpack_hi(u) + unpack_hi(b), 0.0) * unpack_hi(w)
        srow = jnp.sum(t_lo + t_hi, axis=1) + b2_ref[0, 0]
        out_ref[0, 0] = 1.0 + 4.0 / (1.0 + jnp.exp(-srow))

    out = pl.pallas_call(
        body,
        grid=(grid,),
        in_specs=[
            pl.BlockSpec((be, hw), lambda i: (i, 0)),
            pl.BlockSpec((be, hw), lambda i: (i, 0)),
            _full((1, hw)), _full((1, 1)),
        ],
        out_specs=pl.BlockSpec((1, 1, be), lambda i: (i, 0, 0)),
        out_shape=jax.ShapeDtypeStruct((grid, 1, be), F32),
    )(ug, bg, wp2, bp2.reshape(1, 1))
    return out.reshape(e)


def kernel(x_user, x_book, edge_index,
           Wu1, bu1, Wu2, bu2, Wb1, bb1, Wb2, bb2,
           c1b_Wl, c1b_bl, c1b_Wr, c1u_Wl, c1u_bl, c1u_Wr,
           c2b_Wl, c2b_bl, c2b_Wr, c2u_Wl, c2u_bl, c2u_Wr,
           Wp1, bp1, Wp2, bp2):
    nu = x_user.shape[0]
    nb = x_book.shape[0]
    e = edge_index.shape[1]
    h = Wu1.shape[1]

    src = edge_index[0]
    dst = edge_index[1]
    src_all = jnp.concatenate([src, src + nu])
    dst_all = jnp.concatenate([dst, dst + nb])
    z128 = jnp.zeros((max(nu, nb), 128), F32)
    z16 = jnp.zeros((max(nu, nb), 16), F32)
    ones16 = jnp.ones((CH, 16), F32)

    uh0, sig = _encoder(x_user, Wu1, bu1, Wu2, bu2, True)
    bh0, _ = _encoder(x_book, Wb1, bb1, Wb2, bb2, False)

    seg_c = _make_segsum(nu, nb, e, True)
    sum_b, cnt_b = seg_c(uh0, src_all, dst, z128, z16, ones16)
    seg_u = _make_segsum(nb, nu, e, True)
    sum_u, cnt_u = seg_u(bh0, dst_all, src, z128, z16, ones16)
    bh1 = _combine(sum_b, cnt_b, bh0, c1b_Wl, c1b_bl, c1b_Wr, None)
    uh1 = _combine(sum_u, cnt_u, uh0, c1u_Wl, c1u_bl, c1u_Wr, sig)

    seg2_c = _make_segsum(nu, nb, e, False)
    sum_b2 = seg2_c(uh1, src_all, dst, z128, z16, ones16)
    seg2_u = _make_segsum(nb, nu, e, False)
    sum_u2 = seg2_u(bh1, dst_all, src, z128, z16, ones16)
    bf = _combine(sum_b2, cnt_b, bh1, c2b_Wl, c2b_bl, c2b_Wr, None)
    uf = _combine(sum_u2, cnt_u, uh1, c2u_Wl, c2u_bl, c2u_Wr, sig)

    U = _pred_pre(uf, Wp1[:h], bp1)
    B = _pred_pre(bf, Wp1[h:], None)
    wb = Wp2.reshape(h).astype(jnp.bfloat16)
    wlo = lax.convert_element_type(
        lax.bitcast_convert_type(wb[:h // 2], jnp.uint16), jnp.int32)
    whi = lax.convert_element_type(
        lax.bitcast_convert_type(wb[h // 2:], jnp.uint16), jnp.int32)
    wpk = (wlo | (whi << 16)).reshape(1, h // 2)

    ug, bg = _make_edgegather(e, h)(U, B, src, dst)
    return _edge_reduce(ug, bg, wpk, bp2)


# speedup vs baseline: 1.2931x; 1.0001x over previous
import functools

import jax
import jax.numpy as jnp
from jax import lax
from jax.experimental import pallas as pl
from jax.experimental.pallas import tpu as pltpu
from jax.experimental.pallas import tpu_sc as plsc

F32 = jnp.float32
NC = 2
NS = 16
CH = 80
BLK = 1000


def _full(shape):
    return pl.BlockSpec(shape, lambda i: tuple(0 for _ in shape))


def _encoder(x, W1, b1, W2, b2, scale_by_sig):
    n, din = x.shape
    h = W1.shape[1]
    grid = n // BLK

    def body(x_ref, W1_ref, b1_ref, W2_ref, b2_ref, out_ref, sig_ref):
        xb = x_ref[...]
        sig = jax.nn.sigmoid(xb[:, din - 1:din] / 10.0)
        h1 = jnp.maximum(
            jnp.dot(xb, W1_ref[...], preferred_element_type=F32) + b1_ref[...],
            0.0)
        if scale_by_sig:
            h1 = h1 * sig
        h2 = jnp.dot(h1, W2_ref[...], preferred_element_type=F32) + b2_ref[...]
        out_ref[0] = h2[:, :128]
        out_ref[1] = h2[:, 128:]
        sig_ref[...] = jnp.broadcast_to(sig, (BLK, 8))

    out, sig = pl.pallas_call(
        body,
        grid=(grid,),
        in_specs=[
            pl.BlockSpec((BLK, din), lambda i: (i, 0)),
            _full((din, h)), _full((1, h)), _full((h, h)), _full((1, h)),
        ],
        out_specs=[
            pl.BlockSpec((2, BLK, 128), lambda i: (0, i, 0)),
            pl.BlockSpec((BLK, 8), lambda i: (i, 0)),
        ],
        out_shape=[
            jax.ShapeDtypeStruct((2, n, 128), F32),
            jax.ShapeDtypeStruct((n, 8), F32),
        ],
    )(x, W1, b1.reshape(1, h), W2, b2.reshape(1, h))
    return out.reshape(2 * n, 128), sig


def _combine(sum_flat, cnt, xdst_flat, Wl, bl, Wr, sig):
    n = cnt.shape[0]
    h = Wl.shape[1]
    grid = n // BLK
    nb = n // BLK
    with_sig = sig is not None

    def body(*refs):
        if with_sig:
            (s0, s1, cnt_ref, x0, x1, Wl_ref, bl_ref, Wr_ref, sg_ref,
             out_ref) = refs
        else:
            s0, s1, cnt_ref, x0, x1, Wl_ref, bl_ref, Wr_ref, out_ref = refs
        inv = 1.0 / jnp.maximum(cnt_ref[...][:, 0:1], 1.0)
        mean = jnp.concatenate([s0[...], s1[...]], axis=1) * inv
        xd = jnp.concatenate([x0[...], x1[...]], axis=1)
        o = (jnp.dot(mean, Wl_ref[...], preferred_element_type=F32)
             + bl_ref[...]
             + jnp.dot(xd, Wr_ref[...], preferred_element_type=F32))
        a = jnp.maximum(o, 0.0)
        if with_sig:
            a = a * sg_ref[...][:, 0:1]
        out_ref[0] = a[:, :128]
        out_ref[1] = a[:, 128:]

    in_specs = [
        pl.BlockSpec((BLK, 128), lambda i: (i, 0)),
        pl.BlockSpec((BLK, 128), lambda i, nb=nb: (nb + i, 0)),
        pl.BlockSpec((BLK, 16), lambda i: (i, 0)),
        pl.BlockSpec((BLK, 128), lambda i: (i, 0)),
        pl.BlockSpec((BLK, 128), lambda i, nb=nb: (nb + i, 0)),
        _full((h, h)), _full((1, h)), _full((h, h)),
    ]
    args = [sum_flat, sum_flat, cnt, xdst_flat, xdst_flat,
            Wl, bl.reshape(1, h), Wr]
    if with_sig:
        in_specs.append(pl.BlockSpec((BLK, 8), lambda i: (i, 0)))
        args.append(sig)

    out = pl.pallas_call(
        body,
        grid=(grid,),
        in_specs=in_specs,
        out_specs=pl.BlockSpec((2, BLK, 128), lambda i: (0, i, 0)),
        out_shape=jax.ShapeDtypeStruct((2, n, 128), F32),
    )(*args)
    return out.reshape(2 * n, 128)


def _pred_pre(feat_flat, Wp, bp):
    n = feat_flat.shape[0] // 2
    h = Wp.shape[1]
    grid = n // BLK
    nb = n // BLK
    with_b = bp is not None

    def body(*refs):
        if with_b:
            f0, f1, W_ref, b_ref, out_ref = refs
        else:
            f0, f1, W_ref, out_ref = refs
        f = jnp.concatenate([f0[...], f1[...]], axis=1)
        o = jnp.dot(f, W_ref[...], preferred_element_type=F32)
        if with_b:
            o = o + b_ref[...]
        ob = o.astype(jnp.bfloat16)
        lo = lax.convert_element_type(
            lax.bitcast_convert_type(ob[:, :h // 2], jnp.uint16), jnp.int32)
        hi = lax.convert_element_type(
            lax.bitcast_convert_type(ob[:, h // 2:], jnp.uint16), jnp.int32)
        out_ref[...] = lo | (hi << 16)

    in_specs = [
        pl.BlockSpec((BLK, 128), lambda i: (i, 0)),
        pl.BlockSpec((BLK, 128), lambda i, nb=nb: (nb + i, 0)),
        _full((2 * h if Wp.shape[0] == 2 * h else Wp.shape[0], h)),
    ]
    args = [feat_flat, feat_flat, Wp]
    if with_b:
        in_specs.append(_full((1, h)))
        args.append(bp.reshape(1, h))

    return pl.pallas_call(
        body,
        grid=(grid,),
        in_specs=in_specs,
        out_specs=pl.BlockSpec((BLK, h // 2), lambda i: (i, 0)),
        out_shape=jax.ShapeDtypeStruct((n, h // 2), jnp.int32),
    )(*args)


_MESH = plsc.VectorSubcoreMesh(core_axis_name="c", subcore_axis_name="s")
_SC_PARAMS = pltpu.CompilerParams(use_tc_tiling_on_sc=False,
                                  needs_layout_passes=False)


@functools.lru_cache(maxsize=None)
def _make_segsum(n_src, n_dst, n_edges, with_counts):
    epw = n_edges // NS
    iters = epw // CH
    rps = (n_dst // NS) // 8 * 8
    tail = n_dst - NS * rps

    if with_counts:
        out_type = [jax.ShapeDtypeStruct((2 * n_dst, 128), F32),
                    jax.ShapeDtypeStruct((n_dst, 16), F32)]
    else:
        out_type = jax.ShapeDtypeStruct((2 * n_dst, 128), F32)

    bedge = 4000
    bchunks = bedge // CH
    nblocks = epw // bedge
    scratch = [
        pltpu.VMEM((bedge,), jnp.int32),
        pltpu.VMEM((bedge,), jnp.int32),
        pltpu.VMEM((CH, 128), F32),
        pltpu.VMEM((CH, 128), F32),
        pltpu.VMEM((CH, 16), F32),
        pltpu.VMEM_SHARED((n_dst, 128), F32),
        pltpu.VMEM_SHARED((n_dst, 16), F32),
        pltpu.SemaphoreType.DMA,
        pltpu.SemaphoreType.DMA,
    ]

    def body(table, src_all, dst, z128, z16, ones, *rest):
        if with_counts:
            (out, cnt_out, sidx, didx, rows0, rows1, ones_v, acc, cacc,
             sem0, sem1) = rest
        else:
            (out, sidx, didx, rows0, rows1, ones_v, acc, cacc,
             sem0, sem1) = rest
        c = lax.axis_index("c")
        s = lax.axis_index("s")

        row0 = pl.multiple_of(s * rps, 8)
        pltpu.sync_copy(z128.at[pl.ds(row0, rps)], acc.at[pl.ds(row0, rps)])
        if with_counts:
            pltpu.sync_copy(z16.at[pl.ds(row0, rps)],
                            cacc.at[pl.ds(row0, rps)])
            pltpu.sync_copy(ones, ones_v)
        if tail:
            @pl.when(s == NS - 1)
            def _():
                pltpu.sync_copy(z128.at[pl.ds(NS * rps, tail)],
                                acc.at[pl.ds(NS * rps, tail)])
                if with_counts:
                    pltpu.sync_copy(z16.at[pl.ds(NS * rps, tail)],
                                    cacc.at[pl.ds(NS * rps, tail)])

        src_base = pl.multiple_of(c * n_edges + s * epw, 8)
        dst_base = pl.multiple_of(s * epw, 8)
        plsc.subcore_barrier()

        slots = ((rows0, sem0, rows1, sem1), (rows1, sem1, rows0, sem0))

        def block_step(blk, carry):
            boff = blk * bedge
            pltpu.sync_copy(src_all.at[pl.ds(src_base + boff, bedge)], sidx)
            pltpu.sync_copy(dst.at[pl.ds(dst_base + boff, bedge)], didx)
            pltpu.async_copy(table.at[sidx.at[pl.ds(0, CH)]], rows0, sem0)

            def pair(p, carry2):
                for b, (rb, sb, rn, sn) in enumerate(slots):
                    i = p * 2 + b
                    ioff = pl.multiple_of(i * CH, 8)

                    @pl.when(i + 1 < bchunks)
                    def _():
                        noff = pl.multiple_of((i + 1) * CH, 8)
                        pltpu.async_copy(
                            table.at[sidx.at[pl.ds(noff, CH)]], rn, sn)

                    pltpu.make_async_copy(
                        table.at[sidx.at[pl.ds(ioff, CH)]], rb, sb).wait()
                    pltpu.sync_copy(rb, acc.at[didx.at[pl.ds(ioff, CH)]],
                                    add=True)
                    if with_counts:
                        @pl.when(c == 0)
                        def _():
                            pltpu.sync_copy(
                                ones_v, cacc.at[didx.at[pl.ds(ioff, CH)]],
                                add=True)
                return carry2

            lax.fori_loop(0, bchunks // 2, pair, 0)
            return carry

        lax.fori_loop(0, nblocks, block_step, 0)
        plsc.subcore_barrier()

        orow0 = pl.multiple_of(c * n_dst + s * rps, 8)
        pltpu.sync_copy(acc.at[pl.ds(row0, rps)], out.at[pl.ds(orow0, rps)])
        if with_counts:
            @pl.when(c == 0)
            def _():
                pltpu.sync_copy(cacc.at[pl.ds(row0, rps)],
                                cnt_out.at[pl.ds(row0, rps)])
        if tail:
            @pl.when(s == NS - 1)
            def _():
                otail = pl.multiple_of(c * n_dst + NS * rps, 8)
                pltpu.sync_copy(acc.at[pl.ds(NS * rps, tail)],
                                out.at[pl.ds(otail, tail)])
                if with_counts:
                    @pl.when(c == 0)
                    def _():
                        pltpu.sync_copy(cacc.at[pl.ds(NS * rps, tail)],
                                        cnt_out.at[pl.ds(NS * rps, tail)])

    return pl.kernel(body, out_type=out_type, mesh=_MESH,
                     scratch_types=scratch, compiler_params=_SC_PARAMS)


@functools.lru_cache(maxsize=None)
def _make_edgegather(n_edges, h):
    epw = n_edges // (NC * NS)
    iters = epw // CH

    hw = h // 2
    scratch = [
        pltpu.VMEM((epw,), jnp.int32),
        pltpu.VMEM((epw,), jnp.int32),
        pltpu.VMEM((CH, hw), jnp.int32),
        pltpu.VMEM((CH, hw), jnp.int32),
        pltpu.VMEM((CH, hw), jnp.int32),
        pltpu.VMEM((CH, hw), jnp.int32),
        pltpu.SemaphoreType.DMA,
        pltpu.SemaphoreType.DMA,
        pltpu.SemaphoreType.DMA,
        pltpu.SemaphoreType.DMA,
    ]

    def body(U, B, src, dst, u_out, b_out,
             sidx, didx, u0, u1, b0, b1, su0, su1, sb0, sb1):
        c = lax.axis_index("c")
        s = lax.axis_index("s")
        base = pl.multiple_of((c * NS + s) * epw, 8)

        pltpu.sync_copy(src.at[pl.ds(base, epw)], sidx)
        pltpu.sync_copy(dst.at[pl.ds(base, epw)], didx)

        pltpu.async_copy(U.at[sidx.at[pl.ds(0, CH)]], u0, su0)
        pltpu.async_copy(B.at[didx.at[pl.ds(0, CH)]], b0, sb0)

        slots = ((u0, su0, b0, sb0, u1, su1, b1, sb1),
                 (u1, su1, b1, sb1, u0, su0, b0, sb0))

        def pair(p, carry):
            for b, (ub, us, bb, bs, un, usn, bn, bsn) in enumerate(slots):
                i = p * 2 + b

                @pl.when(i < iters)
                def _():
                    ioff = pl.multiple_of(i * CH, 8)

                    @pl.when(i + 1 < iters)
                    def _():
                        noff = pl.multiple_of((i + 1) * CH, 8)
                        pltpu.async_copy(
                            U.at[sidx.at[pl.ds(noff, CH)]], un, usn)
                        pltpu.async_copy(
                            B.at[didx.at[pl.ds(noff, CH)]], bn, bsn)

                    pltpu.make_async_copy(
                        U.at[sidx.at[pl.ds(ioff, CH)]], ub, us).wait()
                    pltpu.make_async_copy(
                        B.at[didx.at[pl.ds(ioff, CH)]], bb, bs).wait()
                    ooff = pl.multiple_of(base + ioff, 8)
                    pltpu.sync_copy(ub, u_out.at[pl.ds(ooff, CH)])
                    pltpu.sync_copy(bb, b_out.at[pl.ds(ooff, CH)])
            return carry

        lax.fori_loop(0, (iters + 1) // 2, pair, 0)

    ot = jax.ShapeDtypeStruct((n_edges, h // 2), jnp.int32)
    return pl.kernel(body, out_type=[ot, ot],
                     mesh=_MESH, scratch_types=scratch)


def _edge_reduce(ug, bg, wlo, whi, bp2):
    e, hw = ug.shape
    be = 3200
    grid = e // be

    def unpack_lo(x):
        return lax.bitcast_convert_type(x << 16, F32)

    def unpack_hi(x):
        return lax.bitcast_convert_type(x & jnp.int32(-65536), F32)

    def body(u_ref, b_ref, wl_ref, wh_ref, b2_ref, out_ref):
        u = u_ref[...]
        b = b_ref[...]
        t_lo = jnp.maximum(unpack_lo(u) + unpack_lo(b), 0.0)
        t_hi = jnp.maximum(unpack_hi(u) + unpack_hi(b), 0.0)
        s = (jnp.dot(t_lo, wl_ref[...], preferred_element_type=F32)
             + jnp.dot(t_hi, wh_ref[...], preferred_element_type=F32))
        srow = s[:, 0] + b2_ref[0, 0]
        out_ref[0, 0] = 1.0 + 4.0 / (1.0 + jnp.exp(-srow))

    out = pl.pallas_call(
        body,
        grid=(grid,),
        in_specs=[
            pl.BlockSpec((be, hw), lambda i: (i, 0)),
            pl.BlockSpec((be, hw), lambda i: (i, 0)),
            _full((hw, 1)), _full((hw, 1)), _full((1, 1)),
        ],
        out_specs=pl.BlockSpec((1, 1, be), lambda i: (i, 0, 0)),
        out_shape=jax.ShapeDtypeStruct((grid, 1, be), F32),
    )(ug, bg, wlo.reshape(hw, 1), whi.reshape(hw, 1), bp2.reshape(1, 1))
    return out.reshape(e)


def kernel(x_user, x_book, edge_index,
           Wu1, bu1, Wu2, bu2, Wb1, bb1, Wb2, bb2,
           c1b_Wl, c1b_bl, c1b_Wr, c1u_Wl, c1u_bl, c1u_Wr,
           c2b_Wl, c2b_bl, c2b_Wr, c2u_Wl, c2u_bl, c2u_Wr,
           Wp1, bp1, Wp2, bp2):
    nu = x_user.shape[0]
    nb = x_book.shape[0]
    e = edge_index.shape[1]
    h = Wu1.shape[1]

    src = edge_index[0]
    dst = edge_index[1]
    src_all = jnp.concatenate([src, src + nu])
    dst_all = jnp.concatenate([dst, dst + nb])
    z128 = jnp.zeros((max(nu, nb), 128), F32)
    z16 = jnp.zeros((max(nu, nb), 16), F32)
    ones16 = jnp.ones((CH, 16), F32)

    uh0, sig = _encoder(x_user, Wu1, bu1, Wu2, bu2, True)
    bh0, _ = _encoder(x_book, Wb1, bb1, Wb2, bb2, False)

    seg_c = _make_segsum(nu, nb, e, True)
    sum_b, cnt_b = seg_c(uh0, src_all, dst, z128, z16, ones16)
    seg_u = _make_segsum(nb, nu, e, True)
    sum_u, cnt_u = seg_u(bh0, dst_all, src, z128, z16, ones16)
    bh1 = _combine(sum_b, cnt_b, bh0, c1b_Wl, c1b_bl, c1b_Wr, None)
    uh1 = _combine(sum_u, cnt_u, uh0, c1u_Wl, c1u_bl, c1u_Wr, sig)

    seg2_c = _make_segsum(nu, nb, e, False)
    sum_b2 = seg2_c(uh1, src_all, dst, z128, z16, ones16)
    seg2_u = _make_segsum(nb, nu, e, False)
    sum_u2 = seg2_u(bh1, dst_all, src, z128, z16, ones16)
    bf = _combine(sum_b2, cnt_b, bh1, c2b_Wl, c2b_bl, c2b_Wr, None)
    uf = _combine(sum_u2, cnt_u, uh1, c2u_Wl, c2u_bl, c2u_Wr, sig)

    U = _pred_pre(uf, Wp1[:h], bp1)
    B = _pred_pre(bf, Wp1[h:], None)

    ug, bg = _make_edgegather(e, h)(U, B, src, dst)
    w2 = Wp2.reshape(h)
    return _edge_reduce(ug, bg, w2[:h // 2], w2[h // 2:], bp2)


# speedup vs baseline: 1.2945x; 1.0010x over previous
import functools

import jax
import jax.numpy as jnp
from jax import lax
from jax.experimental import pallas as pl
from jax.experimental.pallas import tpu as pltpu
from jax.experimental.pallas import tpu_sc as plsc

F32 = jnp.float32
NC = 2
NS = 16
CH = 80
BLK = 1000


def _full(shape):
    return pl.BlockSpec(shape, lambda i: tuple(0 for _ in shape))


def _encoder(x, W1, b1, W2, b2, scale_by_sig):
    n, din = x.shape
    h = W1.shape[1]
    grid = n // BLK

    def body(x_ref, W1_ref, b1_ref, W2_ref, b2_ref, out_ref, sig_ref):
        xb = x_ref[...]
        sig = jax.nn.sigmoid(xb[:, din - 1:din] / 10.0)
        h1 = jnp.maximum(
            jnp.dot(xb.astype(jnp.bfloat16),
                    W1_ref[...].astype(jnp.bfloat16),
                    preferred_element_type=F32) + b1_ref[...],
            0.0)
        if scale_by_sig:
            h1 = h1 * sig
        h2 = jnp.dot(h1.astype(jnp.bfloat16),
                     W2_ref[...].astype(jnp.bfloat16),
                     preferred_element_type=F32) + b2_ref[...]
        out_ref[0] = h2[:, :128]
        out_ref[1] = h2[:, 128:]
        sig_ref[...] = jnp.broadcast_to(sig, (BLK, 8))

    out, sig = pl.pallas_call(
        body,
        grid=(grid,),
        in_specs=[
            pl.BlockSpec((BLK, din), lambda i: (i, 0)),
            _full((din, h)), _full((1, h)), _full((h, h)), _full((1, h)),
        ],
        out_specs=[
            pl.BlockSpec((2, BLK, 128), lambda i: (0, i, 0)),
            pl.BlockSpec((BLK, 8), lambda i: (i, 0)),
        ],
        out_shape=[
            jax.ShapeDtypeStruct((2, n, 128), F32),
            jax.ShapeDtypeStruct((n, 8), F32),
        ],
    )(x, W1, b1.reshape(1, h), W2, b2.reshape(1, h))
    return out.reshape(2 * n, 128), sig


def _combine(sum_flat, cnt, xdst_flat, Wl, bl, Wr, sig):
    n = cnt.shape[0]
    h = Wl.shape[1]
    grid = n // BLK
    nb = n // BLK
    with_sig = sig is not None

    def body(*refs):
        if with_sig:
            (s0, s1, cnt_ref, x0, x1, Wl_ref, bl_ref, Wr_ref, sg_ref,
             out_ref) = refs
        else:
            s0, s1, cnt_ref, x0, x1, Wl_ref, bl_ref, Wr_ref, out_ref = refs
        inv = 1.0 / jnp.maximum(cnt_ref[...][:, 0:1], 1.0)
        mean = jnp.concatenate([s0[...], s1[...]], axis=1) * inv
        xd = jnp.concatenate([x0[...], x1[...]], axis=1)
        o = (jnp.dot(mean.astype(jnp.bfloat16),
                     Wl_ref[...].astype(jnp.bfloat16),
                     preferred_element_type=F32)
             + bl_ref[...]
             + jnp.dot(xd.astype(jnp.bfloat16),
                       Wr_ref[...].astype(jnp.bfloat16),
                       preferred_element_type=F32))
        a = jnp.maximum(o, 0.0)
        if with_sig:
            a = a * sg_ref[...][:, 0:1]
        out_ref[0] = a[:, :128]
        out_ref[1] = a[:, 128:]

    in_specs = [
        pl.BlockSpec((BLK, 128), lambda i: (i, 0)),
        pl.BlockSpec((BLK, 128), lambda i, nb=nb: (nb + i, 0)),
        pl.BlockSpec((BLK, 16), lambda i: (i, 0)),
        pl.BlockSpec((BLK, 128), lambda i: (i, 0)),
        pl.BlockSpec((BLK, 128), lambda i, nb=nb: (nb + i, 0)),
        _full((h, h)), _full((1, h)), _full((h, h)),
    ]
    args = [sum_flat, sum_flat, cnt, xdst_flat, xdst_flat,
            Wl, bl.reshape(1, h), Wr]
    if with_sig:
        in_specs.append(pl.BlockSpec((BLK, 8), lambda i: (i, 0)))
        args.append(sig)

    out = pl.pallas_call(
        body,
        grid=(grid,),
        in_specs=in_specs,
        out_specs=pl.BlockSpec((2, BLK, 128), lambda i: (0, i, 0)),
        out_shape=jax.ShapeDtypeStruct((2, n, 128), F32),
    )(*args)
    return out.reshape(2 * n, 128)


def _pred_pre(feat_flat, Wp, bp):
    n = feat_flat.shape[0] // 2
    h = Wp.shape[1]
    grid = n // BLK
    nb = n // BLK
    with_b = bp is not None

    def body(*refs):
        if with_b:
            f0, f1, W_ref, b_ref, out_ref = refs
        else:
            f0, f1, W_ref, out_ref = refs
        f = jnp.concatenate([f0[...], f1[...]], axis=1)
        o = jnp.dot(f.astype(jnp.bfloat16), W_ref[...].astype(jnp.bfloat16),
                    preferred_element_type=F32)
        if with_b:
            o = o + b_ref[...]
        ob = o.astype(jnp.bfloat16)
        lo = lax.convert_element_type(
            lax.bitcast_convert_type(ob[:, :h // 2], jnp.uint16), jnp.int32)
        hi = lax.convert_element_type(
            lax.bitcast_convert_type(ob[:, h // 2:], jnp.uint16), jnp.int32)
        out_ref[...] = lo | (hi << 16)

    in_specs = [
        pl.BlockSpec((BLK, 128), lambda i: (i, 0)),
        pl.BlockSpec((BLK, 128), lambda i, nb=nb: (nb + i, 0)),
        _full((2 * h if Wp.shape[0] == 2 * h else Wp.shape[0], h)),
    ]
    args = [feat_flat, feat_flat, Wp]
    if with_b:
        in_specs.append(_full((1, h)))
        args.append(bp.reshape(1, h))

    return pl.pallas_call(
        body,
        grid=(grid,),
        in_specs=in_specs,
        out_specs=pl.BlockSpec((BLK, h // 2), lambda i: (i, 0)),
        out_shape=jax.ShapeDtypeStruct((n, h // 2), jnp.int32),
    )(*args)


_MESH = plsc.VectorSubcoreMesh(core_axis_name="c", subcore_axis_name="s")
_SC_PARAMS = pltpu.CompilerParams(use_tc_tiling_on_sc=False,
                                  needs_layout_passes=False)


@functools.lru_cache(maxsize=None)
def _make_segsum(n_src, n_dst, n_edges, with_counts):
    epw = n_edges // NS
    iters = epw // CH
    rps = (n_dst // NS) // 8 * 8
    tail = n_dst - NS * rps

    if with_counts:
        out_type = [jax.ShapeDtypeStruct((2 * n_dst, 128), F32),
                    jax.ShapeDtypeStruct((n_dst, 16), F32)]
    else:
        out_type = jax.ShapeDtypeStruct((2 * n_dst, 128), F32)

    bedge = 4000
    bchunks = bedge // CH
    nblocks = epw // bedge
    scratch = [
        pltpu.VMEM((bedge,), jnp.int32),
        pltpu.VMEM((bedge,), jnp.int32),
        pltpu.VMEM((CH, 128), F32),
        pltpu.VMEM((CH, 128), F32),
        pltpu.VMEM((CH, 16), F32),
        pltpu.VMEM_SHARED((n_dst, 128), F32),
        pltpu.VMEM_SHARED((n_dst, 16), F32),
        pltpu.SemaphoreType.DMA,
        pltpu.SemaphoreType.DMA,
    ]

    def body(table, src_all, dst, z128, z16, ones, *rest):
        if with_counts:
            (out, cnt_out, sidx, didx, rows0, rows1, ones_v, acc, cacc,
             sem0, sem1) = rest
        else:
            (out, sidx, didx, rows0, rows1, ones_v, acc, cacc,
             sem0, sem1) = rest
        c = lax.axis_index("c")
        s = lax.axis_index("s")

        row0 = pl.multiple_of(s * rps, 8)
        pltpu.sync_copy(z128.at[pl.ds(row0, rps)], acc.at[pl.ds(row0, rps)])
        if with_counts:
            pltpu.sync_copy(z16.at[pl.ds(row0, rps)],
                            cacc.at[pl.ds(row0, rps)])
            pltpu.sync_copy(ones, ones_v)
        if tail:
            @pl.when(s == NS - 1)
            def _():
                pltpu.sync_copy(z128.at[pl.ds(NS * rps, tail)],
                                acc.at[pl.ds(NS * rps, tail)])
                if with_counts:
                    pltpu.sync_copy(z16.at[pl.ds(NS * rps, tail)],
                                    cacc.at[pl.ds(NS * rps, tail)])

        src_base = pl.multiple_of(c * n_edges + s * epw, 8)
        dst_base = pl.multiple_of(s * epw, 8)
        plsc.subcore_barrier()

        slots = ((rows0, sem0, rows1, sem1), (rows1, sem1, rows0, sem0))

        def block_step(blk, carry):
            boff = blk * bedge
            pltpu.sync_copy(src_all.at[pl.ds(src_base + boff, bedge)], sidx)
            pltpu.sync_copy(dst.at[pl.ds(dst_base + boff, bedge)], didx)
            pltpu.async_copy(table.at[sidx.at[pl.ds(0, CH)]], rows0, sem0)

            def pair(p, carry2):
                for b, (rb, sb, rn, sn) in enumerate(slots):
                    i = p * 2 + b
                    ioff = pl.multiple_of(i * CH, 8)

                    @pl.when(i + 1 < bchunks)
                    def _():
                        noff = pl.multiple_of((i + 1) * CH, 8)
                        pltpu.async_copy(
                            table.at[sidx.at[pl.ds(noff, CH)]], rn, sn)

                    pltpu.make_async_copy(
                        table.at[sidx.at[pl.ds(ioff, CH)]], rb, sb).wait()
                    pltpu.sync_copy(rb, acc.at[didx.at[pl.ds(ioff, CH)]],
                                    add=True)
                    if with_counts:
                        @pl.when(c == 0)
                        def _():
                            pltpu.sync_copy(
                                ones_v, cacc.at[didx.at[pl.ds(ioff, CH)]],
                                add=True)
                return carry2

            lax.fori_loop(0, bchunks // 2, pair, 0)
            return carry

        lax.fori_loop(0, nblocks, block_step, 0)
        plsc.subcore_barrier()

        orow0 = pl.multiple_of(c * n_dst + s * rps, 8)
        pltpu.sync_copy(acc.at[pl.ds(row0, rps)], out.at[pl.ds(orow0, rps)])
        if with_counts:
            @pl.when(c == 0)
            def _():
                pltpu.sync_copy(cacc.at[pl.ds(row0, rps)],
                                cnt_out.at[pl.ds(row0, rps)])
        if tail:
            @pl.when(s == NS - 1)
            def _():
                otail = pl.multiple_of(c * n_dst + NS * rps, 8)
                pltpu.sync_copy(acc.at[pl.ds(NS * rps, tail)],
                                out.at[pl.ds(otail, tail)])
                if with_counts:
                    @pl.when(c == 0)
                    def _():
                        pltpu.sync_copy(cacc.at[pl.ds(NS * rps, tail)],
                                        cnt_out.at[pl.ds(NS * rps, tail)])

    return pl.kernel(body, out_type=out_type, mesh=_MESH,
                     scratch_types=scratch, compiler_params=_SC_PARAMS)


@functools.lru_cache(maxsize=None)
def _make_edgegather(n_edges, h):
    epw = n_edges // (NC * NS)
    iters = epw // CH

    hw = h // 2
    scratch = [
        pltpu.VMEM((epw,), jnp.int32),
        pltpu.VMEM((epw,), jnp.int32),
        pltpu.VMEM((CH, hw), jnp.int32),
        pltpu.VMEM((CH, hw), jnp.int32),
        pltpu.VMEM((CH, hw), jnp.int32),
        pltpu.VMEM((CH, hw), jnp.int32),
        pltpu.SemaphoreType.DMA,
        pltpu.SemaphoreType.DMA,
        pltpu.SemaphoreType.DMA,
        pltpu.SemaphoreType.DMA,
    ]

    def body(U, B, src, dst, u_out, b_out,
             sidx, didx, u0, u1, b0, b1, su0, su1, sb0, sb1):
        c = lax.axis_index("c")
        s = lax.axis_index("s")
        base = pl.multiple_of((c * NS + s) * epw, 8)

        pltpu.sync_copy(src.at[pl.ds(base, epw)], sidx)
        pltpu.sync_copy(dst.at[pl.ds(base, epw)], didx)

        pltpu.async_copy(U.at[sidx.at[pl.ds(0, CH)]], u0, su0)
        pltpu.async_copy(B.at[didx.at[pl.ds(0, CH)]], b0, sb0)

        slots = ((u0, su0, b0, sb0, u1, su1, b1, sb1),
                 (u1, su1, b1, sb1, u0, su0, b0, sb0))

        def pair(p, carry):
            for b, (ub, us, bb, bs, un, usn, bn, bsn) in enumerate(slots):
                i = p * 2 + b

                @pl.when(i < iters)
                def _():
                    ioff = pl.multiple_of(i * CH, 8)

                    @pl.when(i + 1 < iters)
                    def _():
                        noff = pl.multiple_of((i + 1) * CH, 8)
                        pltpu.async_copy(
                            U.at[sidx.at[pl.ds(noff, CH)]], un, usn)
                        pltpu.async_copy(
                            B.at[didx.at[pl.ds(noff, CH)]], bn, bsn)

                    pltpu.make_async_copy(
                        U.at[sidx.at[pl.ds(ioff, CH)]], ub, us).wait()
                    pltpu.make_async_copy(
                        B.at[didx.at[pl.ds(ioff, CH)]], bb, bs).wait()
                    ooff = pl.multiple_of(base + ioff, 8)
                    pltpu.sync_copy(ub, u_out.at[pl.ds(ooff, CH)])
                    pltpu.sync_copy(bb, b_out.at[pl.ds(ooff, CH)])
            return carry

        lax.fori_loop(0, (iters + 1) // 2, pair, 0)

    ot = jax.ShapeDtypeStruct((n_edges, h // 2), jnp.int32)
    return pl.kernel(body, out_type=[ot, ot],
                     mesh=_MESH, scratch_types=scratch)


def _edge_reduce(ug, bg, wlo, whi, bp2):
    e, hw = ug.shape
    be = 3200
    grid = e // be

    def unpack_lo(x):
        return lax.bitcast_convert_type(x << 16, F32)

    def unpack_hi(x):
        return lax.bitcast_convert_type(x & jnp.int32(-65536), F32)

    def body(u_ref, b_ref, wl_ref, wh_ref, b2_ref, out_ref):
        u = u_ref[...]
        b = b_ref[...]
        t_lo = jnp.maximum(unpack_lo(u) + unpack_lo(b), 0.0)
        t_hi = jnp.maximum(unpack_hi(u) + unpack_hi(b), 0.0)
        s = (jnp.dot(t_lo, wl_ref[...], preferred_element_type=F32)
             + jnp.dot(t_hi, wh_ref[...], preferred_element_type=F32))
        srow = s[:, 0] + b2_ref[0, 0]
        out_ref[0, 0] = 1.0 + 4.0 / (1.0 + jnp.exp(-srow))

    out = pl.pallas_call(
        body,
        grid=(grid,),
        in_specs=[
            pl.BlockSpec((be, hw), lambda i: (i, 0)),
            pl.BlockSpec((be, hw), lambda i: (i, 0)),
            _full((hw, 1)), _full((hw, 1)), _full((1, 1)),
        ],
        out_specs=pl.BlockSpec((1, 1, be), lambda i: (i, 0, 0)),
        out_shape=jax.ShapeDtypeStruct((grid, 1, be), F32),
    )(ug, bg, wlo.reshape(hw, 1), whi.reshape(hw, 1), bp2.reshape(1, 1))
    return out.reshape(e)


def kernel(x_user, x_book, edge_index,
           Wu1, bu1, Wu2, bu2, Wb1, bb1, Wb2, bb2,
           c1b_Wl, c1b_bl, c1b_Wr, c1u_Wl, c1u_bl, c1u_Wr,
           c2b_Wl, c2b_bl, c2b_Wr, c2u_Wl, c2u_bl, c2u_Wr,
           Wp1, bp1, Wp2, bp2):
    nu = x_user.shape[0]
    nb = x_book.shape[0]
    e = edge_index.shape[1]
    h = Wu1.shape[1]

    src = edge_index[0]
    dst = edge_index[1]
    src_all = jnp.concatenate([src, src + nu])
    dst_all = jnp.concatenate([dst, dst + nb])
    z128 = jnp.zeros((max(nu, nb), 128), F32)
    z16 = jnp.zeros((max(nu, nb), 16), F32)
    ones16 = jnp.ones((CH, 16), F32)

    uh0, sig = _encoder(x_user, Wu1, bu1, Wu2, bu2, True)
    bh0, _ = _encoder(x_book, Wb1, bb1, Wb2, bb2, False)

    seg_c = _make_segsum(nu, nb, e, True)
    sum_b, cnt_b = seg_c(uh0, src_all, dst, z128, z16, ones16)
    seg_u = _make_segsum(nb, nu, e, True)
    sum_u, cnt_u = seg_u(bh0, dst_all, src, z128, z16, ones16)
    bh1 = _combine(sum_b, cnt_b, bh0, c1b_Wl, c1b_bl, c1b_Wr, None)
    uh1 = _combine(sum_u, cnt_u, uh0, c1u_Wl, c1u_bl, c1u_Wr, sig)

    seg2_c = _make_segsum(nu, nb, e, False)
    sum_b2 = seg2_c(uh1, src_all, dst, z128, z16, ones16)
    seg2_u = _make_segsum(nb, nu, e, False)
    sum_u2 = seg2_u(bh1, dst_all, src, z128, z16, ones16)
    bf = _combine(sum_b2, cnt_b, bh1, c2b_Wl, c2b_bl, c2b_Wr, None)
    uf = _combine(sum_u2, cnt_u, uh1, c2u_Wl, c2u_bl, c2u_Wr, sig)

    U = _pred_pre(uf, Wp1[:h], bp1)
    B = _pred_pre(bf, Wp1[h:], None)

    ug, bg = _make_edgegather(e, h)(U, B, src, dst)
    w2 = Wp2.reshape(h)
    return _edge_reduce(ug, bg, w2[:h // 2], w2[h // 2:], bp2)


# speedup vs baseline: 1.2945x; 1.0000x over previous
import functools

import jax
import jax.numpy as jnp
from jax import lax
from jax.experimental import pallas as pl
from jax.experimental.pallas import tpu as pltpu
from jax.experimental.pallas import tpu_sc as plsc

F32 = jnp.float32
NC = 2
NS = 16
CH = 80
BLK = 1000


def _full(shape):
    return pl.BlockSpec(shape, lambda i: tuple(0 for _ in shape))


def _encoder(x, W1, b1, W2, b2, scale_by_sig):
    n, din = x.shape
    h = W1.shape[1]
    grid = n // BLK

    def body(x_ref, W1_ref, b1_ref, W2_ref, b2_ref, out_ref, sig_ref):
        xb = x_ref[...]
        sig = jax.nn.sigmoid(xb[:, din - 1:din] / 10.0)
        h1 = jnp.maximum(
            jnp.dot(xb.astype(jnp.bfloat16),
                    W1_ref[...].astype(jnp.bfloat16),
                    preferred_element_type=F32) + b1_ref[...],
            0.0)
        if scale_by_sig:
            h1 = h1 * sig
        h2 = jnp.dot(h1.astype(jnp.bfloat16),
                     W2_ref[...].astype(jnp.bfloat16),
                     preferred_element_type=F32) + b2_ref[...]
        out_ref[0] = h2[:, :128]
        out_ref[1] = h2[:, 128:]
        sig_ref[...] = jnp.broadcast_to(sig, (BLK, 8))

    out, sig = pl.pallas_call(
        body,
        grid=(grid,),
        in_specs=[
            pl.BlockSpec((BLK, din), lambda i: (i, 0)),
            _full((din, h)), _full((1, h)), _full((h, h)), _full((1, h)),
        ],
        out_specs=[
            pl.BlockSpec((2, BLK, 128), lambda i: (0, i, 0)),
            pl.BlockSpec((BLK, 8), lambda i: (i, 0)),
        ],
        out_shape=[
            jax.ShapeDtypeStruct((2, n, 128), F32),
            jax.ShapeDtypeStruct((n, 8), F32),
        ],
    )(x, W1, b1.reshape(1, h), W2, b2.reshape(1, h))
    return out.reshape(2 * n, 128), sig


def _combine(sum_flat, cnt, xdst_flat, Wl, bl, Wr, sig):
    n = cnt.shape[0] // 2
    h = Wl.shape[1]
    grid = n // BLK
    nb = n // BLK
    with_sig = sig is not None

    def body(*refs):
        if with_sig:
            (s0, s1, cnt0_ref, cnt1_ref, x0, x1, Wl_ref, bl_ref, Wr_ref,
             sg_ref, out_ref) = refs
        else:
            (s0, s1, cnt0_ref, cnt1_ref, x0, x1, Wl_ref, bl_ref, Wr_ref,
             out_ref) = refs
        cntv = cnt0_ref[...][:, 0:1] + cnt1_ref[...][:, 0:1]
        inv = 1.0 / jnp.maximum(cntv, 1.0)
        mean = jnp.concatenate([s0[...], s1[...]], axis=1) * inv
        xd = jnp.concatenate([x0[...], x1[...]], axis=1)
        o = (jnp.dot(mean.astype(jnp.bfloat16),
                     Wl_ref[...].astype(jnp.bfloat16),
                     preferred_element_type=F32)
             + bl_ref[...]
             + jnp.dot(xd.astype(jnp.bfloat16),
                       Wr_ref[...].astype(jnp.bfloat16),
                       preferred_element_type=F32))
        a = jnp.maximum(o, 0.0)
        if with_sig:
            a = a * sg_ref[...][:, 0:1]
        out_ref[0] = a[:, :128]
        out_ref[1] = a[:, 128:]

    in_specs = [
        pl.BlockSpec((BLK, 128), lambda i: (i, 0)),
        pl.BlockSpec((BLK, 128), lambda i, nb=nb: (nb + i, 0)),
        pl.BlockSpec((BLK, 16), lambda i: (i, 0)),
        pl.BlockSpec((BLK, 16), lambda i, nb=nb: (nb + i, 0)),
        pl.BlockSpec((BLK, 128), lambda i: (i, 0)),
        pl.BlockSpec((BLK, 128), lambda i, nb=nb: (nb + i, 0)),
        _full((h, h)), _full((1, h)), _full((h, h)),
    ]
    args = [sum_flat, sum_flat, cnt, cnt, xdst_flat, xdst_flat,
            Wl, bl.reshape(1, h), Wr]
    if with_sig:
        in_specs.append(pl.BlockSpec((BLK, 8), lambda i: (i, 0)))
        args.append(sig)

    out = pl.pallas_call(
        body,
        grid=(grid,),
        in_specs=in_specs,
        out_specs=pl.BlockSpec((2, BLK, 128), lambda i: (0, i, 0)),
        out_shape=jax.ShapeDtypeStruct((2, n, 128), F32),
    )(*args)
    return out.reshape(2 * n, 128)


def _pred_pre(feat_flat, Wp, bp):
    n = feat_flat.shape[0] // 2
    h = Wp.shape[1]
    grid = n // BLK
    nb = n // BLK
    with_b = bp is not None

    def body(*refs):
        if with_b:
            f0, f1, W_ref, b_ref, out_ref = refs
        else:
            f0, f1, W_ref, out_ref = refs
        f = jnp.concatenate([f0[...], f1[...]], axis=1)
        o = jnp.dot(f.astype(jnp.bfloat16), W_ref[...].astype(jnp.bfloat16),
                    preferred_element_type=F32)
        if with_b:
            o = o + b_ref[...]
        ob = o.astype(jnp.bfloat16)
        lo = lax.convert_element_type(
            lax.bitcast_convert_type(ob[:, :h // 2], jnp.uint16), jnp.int32)
        hi = lax.convert_element_type(
            lax.bitcast_convert_type(ob[:, h // 2:], jnp.uint16), jnp.int32)
        out_ref[...] = lo | (hi << 16)

    in_specs = [
        pl.BlockSpec((BLK, 128), lambda i: (i, 0)),
        pl.BlockSpec((BLK, 128), lambda i, nb=nb: (nb + i, 0)),
        _full((2 * h if Wp.shape[0] == 2 * h else Wp.shape[0], h)),
    ]
    args = [feat_flat, feat_flat, Wp]
    if with_b:
        in_specs.append(_full((1, h)))
        args.append(bp.reshape(1, h))

    return pl.pallas_call(
        body,
        grid=(grid,),
        in_specs=in_specs,
        out_specs=pl.BlockSpec((BLK, h // 2), lambda i: (i, 0)),
        out_shape=jax.ShapeDtypeStruct((n, h // 2), jnp.int32),
    )(*args)


_MESH = plsc.VectorSubcoreMesh(core_axis_name="c", subcore_axis_name="s")
_SC_PARAMS = pltpu.CompilerParams(use_tc_tiling_on_sc=False,
                                  needs_layout_passes=False)


@functools.lru_cache(maxsize=None)
def _make_segsum(n_src, n_dst, n_edges, with_counts):
    epw = n_edges // NS
    iters = epw // CH
    rps = (n_dst // NS) // 8 * 8
    tail = n_dst - NS * rps

    if with_counts:
        out_type = [jax.ShapeDtypeStruct((2 * n_dst, 128), F32),
                    jax.ShapeDtypeStruct((2 * n_dst, 16), F32)]
    else:
        out_type = jax.ShapeDtypeStruct((2 * n_dst, 128), F32)

    bedge = 4000
    bchunks = bedge // CH
    nblocks = epw // bedge
    scratch = [
        pltpu.VMEM((bedge,), jnp.int32),
        pltpu.VMEM((bedge,), jnp.int32),
        pltpu.VMEM((CH, 128), F32),
        pltpu.VMEM((CH, 128), F32),
        pltpu.VMEM((CH, 16), F32),
        pltpu.VMEM_SHARED((n_dst, 128), F32),
        pltpu.VMEM_SHARED((n_dst, 16), F32),
        pltpu.SemaphoreType.DMA,
        pltpu.SemaphoreType.DMA,
    ]

    def body(table, src_all, dst, z128, z16, ones, *rest):
        if with_counts:
            (out, cnt_out, sidx, didx, rows0, rows1, ones_v, acc, cacc,
             sem0, sem1) = rest
        else:
            (out, sidx, didx, rows0, rows1, ones_v, acc, cacc,
             sem0, sem1) = rest
        c = lax.axis_index("c")
        s = lax.axis_index("s")

        row0 = pl.multiple_of(s * rps, 8)
        pltpu.sync_copy(z128.at[pl.ds(row0, rps)], acc.at[pl.ds(row0, rps)])
        if with_counts:
            pltpu.sync_copy(z16.at[pl.ds(row0, rps)],
                            cacc.at[pl.ds(row0, rps)])
            pltpu.sync_copy(ones, ones_v)
        if tail:
            @pl.when(s == NS - 1)
            def _():
                pltpu.sync_copy(z128.at[pl.ds(NS * rps, tail)],
                                acc.at[pl.ds(NS * rps, tail)])
                if with_counts:
                    pltpu.sync_copy(z16.at[pl.ds(NS * rps, tail)],
                                    cacc.at[pl.ds(NS * rps, tail)])

        src_base = pl.multiple_of(c * n_edges + s * epw, 8)
        dst_base = pl.multiple_of(s * epw, 8)
        plsc.subcore_barrier()

        slots = ((rows0, sem0, rows1, sem1), (rows1, sem1, rows0, sem0))

        def block_step(blk, carry):
            boff = blk * bedge
            pltpu.sync_copy(src_all.at[pl.ds(src_base + boff, bedge)], sidx)
            pltpu.sync_copy(dst.at[pl.ds(dst_base + boff, bedge)], didx)
            pltpu.async_copy(table.at[sidx.at[pl.ds(0, CH)]], rows0, sem0)

            def pair(p, carry2):
                for b, (rb, sb, rn, sn) in enumerate(slots):
                    i = p * 2 + b
                    ioff = pl.multiple_of(i * CH, 8)

                    @pl.when(i + 1 < bchunks)
                    def _():
                        noff = pl.multiple_of((i + 1) * CH, 8)
                        pltpu.async_copy(
                            table.at[sidx.at[pl.ds(noff, CH)]], rn, sn)

                    pltpu.make_async_copy(
                        table.at[sidx.at[pl.ds(ioff, CH)]], rb, sb).wait()
                    pltpu.sync_copy(rb, acc.at[didx.at[pl.ds(ioff, CH)]],
                                    add=True)
                    if with_counts:
                        @pl.when(c == b)
                        def _():
                            pltpu.sync_copy(
                                ones_v, cacc.at[didx.at[pl.ds(ioff, CH)]],
                                add=True)
                return carry2

            lax.fori_loop(0, bchunks // 2, pair, 0)
            return carry

        lax.fori_loop(0, nblocks, block_step, 0)
        plsc.subcore_barrier()

        orow0 = pl.multiple_of(c * n_dst + s * rps, 8)
        pltpu.sync_copy(acc.at[pl.ds(row0, rps)], out.at[pl.ds(orow0, rps)])
        if with_counts:
            pltpu.sync_copy(cacc.at[pl.ds(row0, rps)],
                            cnt_out.at[pl.ds(orow0, rps)])
        if tail:
            @pl.when(s == NS - 1)
            def _():
                otail = pl.multiple_of(c * n_dst + NS * rps, 8)
                pltpu.sync_copy(acc.at[pl.ds(NS * rps, tail)],
                                out.at[pl.ds(otail, tail)])
                if with_counts:
                    pltpu.sync_copy(cacc.at[pl.ds(NS * rps, tail)],
                                    cnt_out.at[pl.ds(otail, tail)])

    return pl.kernel(body, out_type=out_type, mesh=_MESH,
                     scratch_types=scratch, compiler_params=_SC_PARAMS)


@functools.lru_cache(maxsize=None)
def _make_edgegather(n_edges, h):
    epw = n_edges // (NC * NS)
    iters = epw // CH

    hw = h // 2
    scratch = [
        pltpu.VMEM((epw,), jnp.int32),
        pltpu.VMEM((epw,), jnp.int32),
        pltpu.VMEM((CH, hw), jnp.int32),
        pltpu.VMEM((CH, hw), jnp.int32),
        pltpu.VMEM((CH, hw), jnp.int32),
        pltpu.VMEM((CH, hw), jnp.int32),
        pltpu.SemaphoreType.DMA,
        pltpu.SemaphoreType.DMA,
        pltpu.SemaphoreType.DMA,
        pltpu.SemaphoreType.DMA,
    ]

    def body(U, B, src, dst, u_out, b_out,
             sidx, didx, u0, u1, b0, b1, su0, su1, sb0, sb1):
        c = lax.axis_index("c")
        s = lax.axis_index("s")
        base = pl.multiple_of((c * NS + s) * epw, 8)

        pltpu.sync_copy(src.at[pl.ds(base, epw)], sidx)
        pltpu.sync_copy(dst.at[pl.ds(base, epw)], didx)

        pltpu.async_copy(U.at[sidx.at[pl.ds(0, CH)]], u0, su0)
        pltpu.async_copy(B.at[didx.at[pl.ds(0, CH)]], b0, sb0)

        slots = ((u0, su0, b0, sb0, u1, su1, b1, sb1),
                 (u1, su1, b1, sb1, u0, su0, b0, sb0))

        def pair(p, carry):
            for b, (ub, us, bb, bs, un, usn, bn, bsn) in enumerate(slots):
                i = p * 2 + b

                @pl.when(i < iters)
                def _():
                    ioff = pl.multiple_of(i * CH, 8)

                    @pl.when(i + 1 < iters)
                    def _():
                        noff = pl.multiple_of((i + 1) * CH, 8)
                        pltpu.async_copy(
                            U.at[sidx.at[pl.ds(noff, CH)]], un, usn)
                        pltpu.async_copy(
                            B.at[didx.at[pl.ds(noff, CH)]], bn, bsn)

                    pltpu.make_async_copy(
                        U.at[sidx.at[pl.ds(ioff, CH)]], ub, us).wait()
                    pltpu.make_async_copy(
                        B.at[didx.at[pl.ds(ioff, CH)]], bb, bs).wait()
                    ooff = pl.multiple_of(base + ioff, 8)
                    pltpu.sync_copy(ub, u_out.at[pl.ds(ooff, CH)])
                    pltpu.sync_copy(bb, b_out.at[pl.ds(ooff, CH)])
            return carry

        lax.fori_loop(0, (iters + 1) // 2, pair, 0)

    ot = jax.ShapeDtypeStruct((n_edges, h // 2), jnp.int32)
    return pl.kernel(body, out_type=[ot, ot],
                     mesh=_MESH, scratch_types=scratch)


def _edge_reduce(ug, bg, wlo, whi, bp2):
    e, hw = ug.shape
    be = 3200
    grid = e // be

    def unpack_lo(x):
        return lax.bitcast_convert_type(x << 16, F32)

    def unpack_hi(x):
        return lax.bitcast_convert_type(x & jnp.int32(-65536), F32)

    def body(u_ref, b_ref, wl_ref, wh_ref, b2_ref, out_ref):
        u = u_ref[...]
        b = b_ref[...]
        t_lo = jnp.maximum(unpack_lo(u) + unpack_lo(b), 0.0)
        t_hi = jnp.maximum(unpack_hi(u) + unpack_hi(b), 0.0)
        s = (jnp.dot(t_lo, wl_ref[...], preferred_element_type=F32)
             + jnp.dot(t_hi, wh_ref[...], preferred_element_type=F32))
        srow = s[:, 0] + b2_ref[0, 0]
        out_ref[0, 0] = 1.0 + 4.0 / (1.0 + jnp.exp(-srow))

    out = pl.pallas_call(
        body,
        grid=(grid,),
        in_specs=[
            pl.BlockSpec((be, hw), lambda i: (i, 0)),
            pl.BlockSpec((be, hw), lambda i: (i, 0)),
            _full((hw, 1)), _full((hw, 1)), _full((1, 1)),
        ],
        out_specs=pl.BlockSpec((1, 1, be), lambda i: (i, 0, 0)),
        out_shape=jax.ShapeDtypeStruct((grid, 1, be), F32),
    )(ug, bg, wlo.reshape(hw, 1), whi.reshape(hw, 1), bp2.reshape(1, 1))
    return out.reshape(e)


def kernel(x_user, x_book, edge_index,
           Wu1, bu1, Wu2, bu2, Wb1, bb1, Wb2, bb2,
           c1b_Wl, c1b_bl, c1b_Wr, c1u_Wl, c1u_bl, c1u_Wr,
           c2b_Wl, c2b_bl, c2b_Wr, c2u_Wl, c2u_bl, c2u_Wr,
           Wp1, bp1, Wp2, bp2):
    nu = x_user.shape[0]
    nb = x_book.shape[0]
    e = edge_index.shape[1]
    h = Wu1.shape[1]

    src = edge_index[0]
    dst = edge_index[1]
    src_all = jnp.concatenate([src, src + nu])
    dst_all = jnp.concatenate([dst, dst + nb])
    z128 = jnp.zeros((max(nu, nb), 128), F32)
    z16 = jnp.zeros((max(nu, nb), 16), F32)
    ones16 = jnp.ones((CH, 16), F32)

    uh0, sig = _encoder(x_user, Wu1, bu1, Wu2, bu2, True)
    bh0, _ = _encoder(x_book, Wb1, bb1, Wb2, bb2, False)

    seg_c = _make_segsum(nu, nb, e, True)
    sum_b, cnt_b = seg_c(uh0, src_all, dst, z128, z16, ones16)
    seg_u = _make_segsum(nb, nu, e, True)
    sum_u, cnt_u = seg_u(bh0, dst_all, src, z128, z16, ones16)
    bh1 = _combine(sum_b, cnt_b, bh0, c1b_Wl, c1b_bl, c1b_Wr, None)
    uh1 = _combine(sum_u, cnt_u, uh0, c1u_Wl, c1u_bl, c1u_Wr, sig)

    seg2_c = _make_segsum(nu, nb, e, False)
    sum_b2 = seg2_c(uh1, src_all, dst, z128, z16, ones16)
    seg2_u = _make_segsum(nb, nu, e, False)
    sum_u2 = seg2_u(bh1, dst_all, src, z128, z16, ones16)
    bf = _combine(sum_b2, cnt_b, bh1, c2b_Wl, c2b_bl, c2b_Wr, None)
    uf = _combine(sum_u2, cnt_u, uh1, c2u_Wl, c2u_bl, c2u_Wr, sig)

    U = _pred_pre(uf, Wp1[:h], bp1)
    B = _pred_pre(bf, Wp1[h:], None)

    ug, bg = _make_edgegather(e, h)(U, B, src, dst)
    w2 = Wp2.reshape(h)
    return _edge_reduce(ug, bg, w2[:h // 2], w2[h // 2:], bp2)
